# baseline (speedup 1.0000x reference)
import functools
import math

import jax
import jax.numpy as jnp
from jax import lax
from jax.experimental import pallas as pl
from jax.experimental.pallas import tpu as pltpu
from jax.experimental.pallas import tpu_sc as plsc

D_MODEL = 64
SCALE = math.sqrt(D_MODEL)
LANES = 16
NC, NS = 2, 16
NW = NC * NS
IDXW = 128
NSUB = 4
CHUNK = NSUB * IDXW


def _emb_body(x_hbm, table_hbm, out_hbm, idx_v, rows_v, gsem):
    wid = lax.axis_index("s") * NC + lax.axis_index("c")
    nrows_w = x_hbm.shape[0] // NW
    nchunk = nrows_w // NSUB
    row0 = wid * nrows_w

    def chunk_body(g, carry):
        base = row0 + g * NSUB
        pltpu.sync_copy(x_hbm.at[pl.ds(base, NSUB), :], idx_v)
        copies = [
            pltpu.async_copy(
                table_hbm.at[idx_v.at[j]],
                rows_v.at[pl.ds(j * IDXW, IDXW), :],
                gsem,
            )
            for j in range(NSUB)
        ]
        for cp in copies:
            cp.wait()

        def scale_body(r, c):
            for cc in range(D_MODEL // LANES):
                sl = (r, pl.ds(cc * LANES, LANES))
                rows_v[sl] = rows_v[sl] * SCALE
            return c

        lax.fori_loop(0, CHUNK, scale_body, 0)
        out0 = base * IDXW
        pltpu.sync_copy(rows_v, out_hbm.at[pl.ds(out0, CHUNK), :])
        return carry

    lax.fori_loop(0, nchunk, chunk_body, 0)


def _make_call(n_rows):
    return functools.partial(
        pl.kernel,
        mesh=plsc.VectorSubcoreMesh(core_axis_name="c", subcore_axis_name="s"),
        out_type=jax.ShapeDtypeStruct((n_rows, D_MODEL), jnp.float32),
        scratch_types=[
            pltpu.VMEM((NSUB, IDXW), jnp.int32),
            pltpu.VMEM((CHUNK, D_MODEL), jnp.float32),
            pltpu.SemaphoreType.DMA,
        ],
        compiler_params=pltpu.CompilerParams(use_tc_tiling_on_sc=False),
    )(_emb_body)


def kernel(x, table):
    s, t = x.shape
    n = s * t
    xf = x.astype(jnp.int32).reshape(n // IDXW, IDXW)
    out = _make_call(n)(xf, table)
    return out.reshape(s, t, D_MODEL)


# speedup vs baseline: 1.1344x; 1.1344x over previous
import math

import jax
import jax.numpy as jnp
from jax import lax
from jax.experimental import pallas as pl
from jax.experimental.pallas import tpu as pltpu
from jax.experimental.pallas import tpu_sc as plsc

D_MODEL = 64
SCALE = math.sqrt(D_MODEL)
LANES = 16
NC, NS = 2, 16
NW = NC * NS
IDXW = 128
NSUB = 4
CHUNK = NSUB * IDXW
NBUF = 3

N_ROWS = 4096 * 200
ROWS_W = N_ROWS // NW
IDXROWS_W = ROWS_W // IDXW
NCHUNK = ROWS_W // CHUNK
NMAIN = (NCHUNK // NBUF) * NBUF


def _emb_body(x_hbm, table_hbm, out_hbm,
              idx_v, r0, r1, r2, g0, g1, g2, s0, s1, s2):
    rows = (r0, r1, r2)
    gsem = (g0, g1, g2)
    ssem = (s0, s1, s2)
    wid = lax.axis_index("s") * NC + lax.axis_index("c")
    irow0 = wid * IDXROWS_W
    out0 = wid * ROWS_W

    pltpu.sync_copy(x_hbm.at[pl.ds(irow0, IDXROWS_W), :], idx_v)

    def fire_gather(g, b):
        for j in range(NSUB):
            pltpu.async_copy(
                table_hbm.at[idx_v.at[g * NSUB + j]],
                rows[b].at[pl.ds(j * IDXW, IDXW), :],
                gsem[b],
            )

    def wait_gather(b):
        pltpu.make_async_copy(
            out_hbm.at[pl.ds(0, CHUNK), :], rows[b], gsem[b]).wait()

    def fire_store(g, b):
        pltpu.async_copy(
            rows[b], out_hbm.at[pl.ds(out0 + g * CHUNK, CHUNK), :], ssem[b])

    def wait_store(b):
        pltpu.make_async_copy(
            rows[b], out_hbm.at[pl.ds(0, CHUNK), :], ssem[b]).wait()

    def scale(b):
        @plsc.parallel_loop(0, CHUNK, 1, unroll=8)
        def _(r):
            for cc in range(D_MODEL // LANES):
                sl = (r, pl.ds(cc * LANES, LANES))
                rows[b][sl] = rows[b][sl] * SCALE

    fire_gather(0, 0)
    fire_gather(1, 1)

    def body(i, carry):
        for db in range(NBUF):
            g = NBUF * i + db
            b = db
            bn = (db + 2) % NBUF
            if db == 0:
                @pl.when(i >= 1)
                def _():
                    wait_store(bn)
            else:
                wait_store(bn)
            fire_gather(g + 2, bn)
            wait_gather(b)
            scale(b)
            fire_store(g, b)
        return carry

    lax.fori_loop(0, NMAIN // NBUF, body, 0)

    for (g, b) in ((NMAIN, 0), (NMAIN + 1, 1)):
        wait_gather(b)
        scale(b)
        fire_store(g, b)
    for b in (2, 0, 1):
        wait_store(b)


_EMB_CALL = pl.kernel(
    _emb_body,
    mesh=plsc.VectorSubcoreMesh(core_axis_name="c", subcore_axis_name="s"),
    out_type=jax.ShapeDtypeStruct((N_ROWS, D_MODEL), jnp.float32),
    scratch_types=[
        pltpu.VMEM((NW * IDXROWS_W // NW, IDXW), jnp.int32),
        pltpu.VMEM((CHUNK, D_MODEL), jnp.float32),
        pltpu.VMEM((CHUNK, D_MODEL), jnp.float32),
        pltpu.VMEM((CHUNK, D_MODEL), jnp.float32),
        pltpu.SemaphoreType.DMA,
        pltpu.SemaphoreType.DMA,
        pltpu.SemaphoreType.DMA,
        pltpu.SemaphoreType.DMA,
        pltpu.SemaphoreType.DMA,
        pltpu.SemaphoreType.DMA,
    ],
    compiler_params=pltpu.CompilerParams(use_tc_tiling_on_sc=False),
)


def kernel(x, table):
    s, t = x.shape
    xf = x.astype(jnp.int32).reshape((s * t) // IDXW, IDXW)
    out = _EMB_CALL(xf, table)
    return out.reshape(s, t, D_MODEL)


# speedup vs baseline: 1.1361x; 1.0015x over previous
import math

import jax
import jax.numpy as jnp
from jax import lax
from jax.experimental import pallas as pl
from jax.experimental.pallas import tpu as pltpu
from jax.experimental.pallas import tpu_sc as plsc

D_MODEL = 64
SCALE = math.sqrt(D_MODEL)
LANES = 16
NC, NS = 2, 16
NW = NC * NS
SEQ, TOK = 4096, 200
ROWS_W = SEQ // NW
NBUF = 3
NMAIN = (ROWS_W // NBUF) * NBUF
GA, GB = 128, TOK - 128


def _emb_body(x_hbm, table_hbm, out_hbm,
              idx_v, r0, r1, r2, g0, g1, g2, s0, s1, s2):
    rows = (r0, r1, r2)
    gsem = (g0, g1, g2)
    ssem = (s0, s1, s2)
    wid = lax.axis_index("s") * NC + lax.axis_index("c")
    xrow0 = wid * ROWS_W

    pltpu.sync_copy(x_hbm.at[pl.ds(xrow0, ROWS_W), :], idx_v)

    def fire_gather(r, b):
        pltpu.async_copy(
            table_hbm.at[idx_v.at[r, pl.ds(0, GA)]],
            rows[b].at[pl.ds(0, GA), :], gsem[b])
        pltpu.async_copy(
            table_hbm.at[idx_v.at[r, pl.ds(GA, GB)]],
            rows[b].at[pl.ds(GA, GB), :], gsem[b])

    def wait_gather(b):
        pltpu.make_async_copy(
            out_hbm.at[0], rows[b], gsem[b]).wait()

    def fire_store(r, b):
        pltpu.async_copy(rows[b], out_hbm.at[xrow0 + r], ssem[b])

    def wait_store(b):
        pltpu.make_async_copy(rows[b], out_hbm.at[0], ssem[b]).wait()

    def scale(b):
        @plsc.parallel_loop(0, TOK, 1, unroll=8)
        def _(t):
            for cc in range(D_MODEL // LANES):
                sl = (t, pl.ds(cc * LANES, LANES))
                rows[b][sl] = rows[b][sl] * SCALE

    fire_gather(0, 0)
    fire_gather(1, 1)

    def body(i, carry):
        for db in range(NBUF):
            r = NBUF * i + db
            b = db
            bn = (db + 2) % NBUF
            if db == 0:
                @pl.when(i >= 1)
                def _():
                    wait_store(bn)
            else:
                wait_store(bn)
            fire_gather(r + 2, bn)
            wait_gather(b)
            scale(b)
            fire_store(r, b)
        return carry

    lax.fori_loop(0, NMAIN // NBUF, body, 0)

    for (r, b) in ((NMAIN, 0), (NMAIN + 1, 1)):
        wait_gather(b)
        scale(b)
        fire_store(r, b)
    for b in (2, 0, 1):
        wait_store(b)


_EMB_CALL = pl.kernel(
    _emb_body,
    mesh=plsc.VectorSubcoreMesh(core_axis_name="c", subcore_axis_name="s"),
    out_type=jax.ShapeDtypeStruct((SEQ, TOK, D_MODEL), jnp.float32),
    scratch_types=[
        pltpu.VMEM((ROWS_W, TOK), jnp.int32),
        pltpu.VMEM((TOK, D_MODEL), jnp.float32),
        pltpu.VMEM((TOK, D_MODEL), jnp.float32),
        pltpu.VMEM((TOK, D_MODEL), jnp.float32),
        pltpu.SemaphoreType.DMA,
        pltpu.SemaphoreType.DMA,
        pltpu.SemaphoreType.DMA,
        pltpu.SemaphoreType.DMA,
        pltpu.SemaphoreType.DMA,
        pltpu.SemaphoreType.DMA,
    ],
    compiler_params=pltpu.CompilerParams(use_tc_tiling_on_sc=False),
)


def kernel(x, table):
    return _EMB_CALL(x.astype(jnp.int32), table)


# speedup vs baseline: 1.4420x; 1.2692x over previous
import math

import jax
import jax.numpy as jnp
from jax import lax
from jax.experimental import pallas as pl
from jax.experimental.pallas import tpu as pltpu
from jax.experimental.pallas import tpu_sc as plsc

VOCAB_N = 1000000
D_MODEL = 64
SCALE = math.sqrt(D_MODEL)
NC, NS = 2, 16
NW = NC * NS
SEQ, TOK = 4096, 200
ROWS_W = SEQ // NW
NBUF = 3
NMAIN = (ROWS_W // NBUF) * NBUF
GA, GB = 128, TOK - 128
VB = 4096


def _k1_body(t_t_ref, out_ref):
    blk = t_t_ref[...]
    eye8 = jnp.eye(D_MODEL, dtype=jnp.float32) * SCALE
    rows = jax.lax.dot_general(
        blk, eye8, (((0,), (0,)), ((), ())),
        precision=jax.lax.Precision.HIGHEST,
        preferred_element_type=jnp.float32)
    out_ref[...] = jnp.concatenate([rows, rows], axis=1)


_K1 = pl.pallas_call(
    _k1_body,
    grid=(pl.cdiv(VOCAB_N, VB),),
    in_specs=[pl.BlockSpec((D_MODEL, VB), lambda i: (0, i))],
    out_specs=pl.BlockSpec((VB, 2 * D_MODEL), lambda i: (i, 0)),
    out_shape=jax.ShapeDtypeStruct((VOCAB_N, 2 * D_MODEL), jnp.float32),
)


def _k2_body(x_hbm, t8_hbm, out_hbm, idx_v, r0, r1, r2, g0, g1, g2, s0, s1, s2):
    rows = (r0, r1, r2)
    gsem = (g0, g1, g2)
    ssem = (s0, s1, s2)
    wid = lax.axis_index("s") * NC + lax.axis_index("c")
    srow0 = wid * ROWS_W

    pltpu.sync_copy(x_hbm.at[pl.ds(srow0, ROWS_W), :], idx_v)

    def fire_gather(s, b):
        pltpu.async_copy(
            t8_hbm.at[idx_v.at[s, pl.ds(0, GA)]],
            rows[b].at[pl.ds(0, GA), :], gsem[b])
        pltpu.async_copy(
            t8_hbm.at[idx_v.at[s, pl.ds(GA, GB)]],
            rows[b].at[pl.ds(GA, GB), :], gsem[b])

    def wait_gather(b):
        pltpu.make_async_copy(
            t8_hbm.at[pl.ds(0, TOK), :], rows[b], gsem[b]).wait()

    def fire_store(s, b):
        pltpu.async_copy(rows[b], out_hbm.at[srow0 + s], ssem[b])

    def wait_store(b):
        pltpu.make_async_copy(rows[b], out_hbm.at[0], ssem[b]).wait()

    fire_gather(0, 0)
    fire_gather(1, 1)

    def body(i, carry):
        for db in range(NBUF):
            s = NBUF * i + db
            b = db
            bn = (db + 2) % NBUF
            if db == 0:
                @pl.when(i >= 1)
                def _():
                    wait_store(bn)
            else:
                wait_store(bn)
            fire_gather(s + 2, bn)
            wait_gather(b)
            fire_store(s, b)
        return carry

    lax.fori_loop(0, NMAIN // NBUF, body, 0)

    for (s, b) in ((NMAIN, 0), (NMAIN + 1, 1)):
        wait_gather(b)
        fire_store(s, b)
    for b in (2, 0, 1):
        wait_store(b)


_K2 = pl.kernel(
    _k2_body,
    mesh=plsc.VectorSubcoreMesh(core_axis_name="c", subcore_axis_name="s"),
    out_type=jax.ShapeDtypeStruct((SEQ, TOK, 2 * D_MODEL), jnp.float32),
    scratch_types=[
        pltpu.VMEM((ROWS_W, TOK), jnp.int32),
        pltpu.VMEM((TOK, 2 * D_MODEL), jnp.float32),
        pltpu.VMEM((TOK, 2 * D_MODEL), jnp.float32),
        pltpu.VMEM((TOK, 2 * D_MODEL), jnp.float32),
        pltpu.SemaphoreType.DMA,
        pltpu.SemaphoreType.DMA,
        pltpu.SemaphoreType.DMA,
        pltpu.SemaphoreType.DMA,
        pltpu.SemaphoreType.DMA,
        pltpu.SemaphoreType.DMA,
    ],
    compiler_params=pltpu.CompilerParams(use_tc_tiling_on_sc=True),
)


def kernel(x, table):
    t8 = _K1(table.T)
    out_wide = _K2(x.astype(jnp.int32), t8)
    return out_wide[:, :, :D_MODEL]


# speedup vs baseline: 1.6080x; 1.1151x over previous
import math

import jax
import jax.numpy as jnp
from jax import lax
from jax.experimental import pallas as pl
from jax.experimental.pallas import tpu as pltpu
from jax.experimental.pallas import tpu_sc as plsc

VOCAB_N = 1000000
D_MODEL = 64
SCALE = math.sqrt(D_MODEL)
NC, NS = 2, 16
NW = NC * NS
SEQ, TOK = 4096, 200
ROWS_W = SEQ // NW
NBUF = 3
NMAIN = (ROWS_W // NBUF) * NBUF
GA, GB = 128, TOK - 128
VB = 4096


def _k1_body(t_t_ref, out_ref):
    rows = jnp.transpose(t_t_ref[...], (1, 0)) * SCALE
    out_ref[...] = jnp.concatenate([rows, rows], axis=1)


_K1 = pl.pallas_call(
    _k1_body,
    grid=(pl.cdiv(VOCAB_N, VB),),
    in_specs=[pl.BlockSpec((D_MODEL, VB), lambda i: (0, i))],
    out_specs=pl.BlockSpec((VB, 2 * D_MODEL), lambda i: (i, 0)),
    out_shape=jax.ShapeDtypeStruct((VOCAB_N, 2 * D_MODEL), jnp.float32),
)


def _k2_body(x_hbm, t8_hbm, out_hbm, idx_v, r0, r1, r2, g0, g1, g2, s0, s1, s2):
    rows = (r0, r1, r2)
    gsem = (g0, g1, g2)
    ssem = (s0, s1, s2)
    wid = lax.axis_index("s") * NC + lax.axis_index("c")
    srow0 = wid * ROWS_W

    pltpu.sync_copy(x_hbm.at[pl.ds(srow0, ROWS_W), :], idx_v)

    def fire_gather(s, b):
        pltpu.async_copy(
            t8_hbm.at[idx_v.at[s, pl.ds(0, GA)]],
            rows[b].at[pl.ds(0, GA), :], gsem[b])
        pltpu.async_copy(
            t8_hbm.at[idx_v.at[s, pl.ds(GA, GB)]],
            rows[b].at[pl.ds(GA, GB), :], gsem[b])

    def wait_gather(b):
        pltpu.make_async_copy(
            t8_hbm.at[pl.ds(0, TOK), :], rows[b], gsem[b]).wait()

    def fire_store(s, b):
        pltpu.async_copy(rows[b], out_hbm.at[srow0 + s], ssem[b])

    def wait_store(b):
        pltpu.make_async_copy(rows[b], out_hbm.at[0], ssem[b]).wait()

    fire_gather(0, 0)
    fire_gather(1, 1)

    def body(i, carry):
        for db in range(NBUF):
            s = NBUF * i + db
            b = db
            bn = (db + 2) % NBUF
            if db == 0:
                @pl.when(i >= 1)
                def _():
                    wait_store(bn)
            else:
                wait_store(bn)
            fire_gather(s + 2, bn)
            wait_gather(b)
            fire_store(s, b)
        return carry

    lax.fori_loop(0, NMAIN // NBUF, body, 0)

    for (s, b) in ((NMAIN, 0), (NMAIN + 1, 1)):
        wait_gather(b)
        fire_store(s, b)
    for b in (2, 0, 1):
        wait_store(b)


_K2 = pl.kernel(
    _k2_body,
    mesh=plsc.VectorSubcoreMesh(core_axis_name="c", subcore_axis_name="s"),
    out_type=jax.ShapeDtypeStruct((SEQ, TOK, 2 * D_MODEL), jnp.float32),
    scratch_types=[
        pltpu.VMEM((ROWS_W, TOK), jnp.int32),
        pltpu.VMEM((TOK, 2 * D_MODEL), jnp.float32),
        pltpu.VMEM((TOK, 2 * D_MODEL), jnp.float32),
        pltpu.VMEM((TOK, 2 * D_MODEL), jnp.float32),
        pltpu.SemaphoreType.DMA,
        pltpu.SemaphoreType.DMA,
        pltpu.SemaphoreType.DMA,
        pltpu.SemaphoreType.DMA,
        pltpu.SemaphoreType.DMA,
        pltpu.SemaphoreType.DMA,
    ],
    compiler_params=pltpu.CompilerParams(use_tc_tiling_on_sc=True),
)


def kernel(x, table):
    t8 = _K1(table.T)
    out_wide = _K2(x.astype(jnp.int32), t8)
    return out_wide[:, :, :D_MODEL]


# speedup vs baseline: 1.9129x; 1.1896x over previous
import math

import jax
import jax.numpy as jnp
from jax import lax
from jax.experimental import pallas as pl
from jax.experimental.pallas import tpu as pltpu
from jax.experimental.pallas import tpu_sc as plsc

VOCAB_N = 1000000
D_MODEL = 64
SCALE = math.sqrt(D_MODEL)
NC, NS = 2, 16
NW = NC * NS
SEQ, TOK = 4096, 200
ROWS_W = SEQ // NW
NBUF = 3
NMAIN = (ROWS_W // NBUF) * NBUF
GA, GB = 128, TOK - 128
VB = 4096


def _k1_body(t_t_ref, out_ref):
    rows = jnp.transpose(t_t_ref[...], (1, 0)) * SCALE
    out_ref[...] = jnp.concatenate([rows, rows], axis=1)


_K1 = pl.pallas_call(
    _k1_body,
    grid=(pl.cdiv(VOCAB_N, VB),),
    in_specs=[pl.BlockSpec((D_MODEL, VB), lambda i: (0, i))],
    out_specs=pl.BlockSpec((VB, 2 * D_MODEL), lambda i: (i, 0)),
    out_shape=jax.ShapeDtypeStruct((VOCAB_N, 2 * D_MODEL), jnp.float32),
)


def _k2_body(x_hbm, t8_hbm, out_hbm, idx_v, r0, r1, r2, g0, g1, g2, s0, s1, s2):
    rows = (r0, r1, r2)
    gsem = (g0, g1, g2)
    ssem = (s0, s1, s2)
    wid = lax.axis_index("s") * NC + lax.axis_index("c")
    srow0 = wid * ROWS_W

    pltpu.sync_copy(x_hbm.at[pl.ds(srow0, ROWS_W), :], idx_v)

    def fire_gather(s, b):
        pltpu.async_copy(
            t8_hbm.at[idx_v.at[s, pl.ds(0, GA)]],
            rows[b].at[pl.ds(0, GA), :], gsem[b])
        pltpu.async_copy(
            t8_hbm.at[idx_v.at[s, pl.ds(GA, GB)]],
            rows[b].at[pl.ds(GA, GB), :], gsem[b])

    def wait_gather(b):
        pltpu.make_async_copy(
            t8_hbm.at[pl.ds(0, TOK), :], rows[b], gsem[b]).wait()

    def fire_store(s, b):
        pltpu.async_copy(
            rows[b], out_hbm.at[srow0 + s, :, pl.ds(0, D_MODEL)], ssem[b])

    def wait_store(b):
        pltpu.make_async_copy(
            rows[b], out_hbm.at[0, :, pl.ds(0, D_MODEL)], ssem[b]).wait()

    fire_gather(0, 0)
    fire_gather(1, 1)

    def body(i, carry):
        for db in range(NBUF):
            s = NBUF * i + db
            b = db
            bn = (db + 2) % NBUF
            if db == 0:
                @pl.when(i >= 1)
                def _():
                    wait_store(bn)
            else:
                wait_store(bn)
            fire_gather(s + 2, bn)
            wait_gather(b)
            fire_store(s, b)
        return carry

    lax.fori_loop(0, NMAIN // NBUF, body, 0)

    for (s, b) in ((NMAIN, 0), (NMAIN + 1, 1)):
        wait_gather(b)
        fire_store(s, b)
    for b in (2, 0, 1):
        wait_store(b)


_K2 = pl.kernel(
    _k2_body,
    mesh=plsc.VectorSubcoreMesh(core_axis_name="c", subcore_axis_name="s"),
    out_type=jax.ShapeDtypeStruct((SEQ, TOK, 2 * D_MODEL), jnp.float32),
    scratch_types=[
        pltpu.VMEM((ROWS_W, TOK), jnp.int32),
        pltpu.VMEM((TOK, D_MODEL), jnp.float32),
        pltpu.VMEM((TOK, D_MODEL), jnp.float32),
        pltpu.VMEM((TOK, D_MODEL), jnp.float32),
        pltpu.SemaphoreType.DMA,
        pltpu.SemaphoreType.DMA,
        pltpu.SemaphoreType.DMA,
        pltpu.SemaphoreType.DMA,
        pltpu.SemaphoreType.DMA,
        pltpu.SemaphoreType.DMA,
    ],
    compiler_params=pltpu.CompilerParams(use_tc_tiling_on_sc=False),
)


def kernel(x, table):
    t8 = _K1(table.T)
    stg = t8.reshape(2 * VOCAB_N, D_MODEL)
    out_wide = _K2(x.astype(jnp.int32) * 2, stg)
    return out_wide[:, :, :D_MODEL]


# speedup vs baseline: 2.3300x; 1.2180x over previous
import math

import jax
import jax.numpy as jnp
from jax import lax
from jax.experimental import pallas as pl
from jax.experimental.pallas import tpu as pltpu
from jax.experimental.pallas import tpu_sc as plsc

VOCAB_N = 1000000
D_MODEL = 64
SCALE = math.sqrt(D_MODEL)
NC, NS = 2, 16
NW = NC * NS
SEQ, TOK = 4096, 200
ROWS_W = SEQ // NW
NBUF = 3
NMAIN = (ROWS_W // NBUF) * NBUF
GA, GB = 128, TOK - 128
VB = 4096


K1_GRID = (VOCAB_N + 2 * VB - 1) // (2 * VB)
P_ROWS = K1_GRID * VB
N_INBLK = (VOCAB_N + VB - 1) // VB - 1


def _k1_body(tl_ref, th_ref, out_ref):
    lo = jnp.transpose(tl_ref[...], (1, 0)) * SCALE
    hi = jnp.transpose(th_ref[...], (1, 0)) * SCALE
    out_ref[...] = jnp.concatenate([lo, hi], axis=1)


_K1 = pl.pallas_call(
    _k1_body,
    grid=(K1_GRID,),
    in_specs=[
        pl.BlockSpec((D_MODEL, VB), lambda i: (0, jnp.minimum(2 * i, N_INBLK))),
        pl.BlockSpec((D_MODEL, VB),
                     lambda i: (0, jnp.minimum(2 * i + 1, N_INBLK))),
    ],
    out_specs=pl.BlockSpec((VB, 2 * D_MODEL), lambda i: (i, 0)),
    out_shape=jax.ShapeDtypeStruct((P_ROWS, 2 * D_MODEL), jnp.float32),
)


def _k2_body(x_hbm, t8_hbm, out_hbm, idx_v, r0, r1, r2, g0, g1, g2, s0, s1, s2):
    rows = (r0, r1, r2)
    gsem = (g0, g1, g2)
    ssem = (s0, s1, s2)
    wid = lax.axis_index("s") * NC + lax.axis_index("c")
    srow0 = wid * ROWS_W

    pltpu.sync_copy(x_hbm.at[pl.ds(srow0, ROWS_W), :], idx_v)

    def fire_gather(s, b):
        pltpu.async_copy(
            t8_hbm.at[idx_v.at[s, pl.ds(0, GA)]],
            rows[b].at[pl.ds(0, GA), :], gsem[b])
        pltpu.async_copy(
            t8_hbm.at[idx_v.at[s, pl.ds(GA, GB)]],
            rows[b].at[pl.ds(GA, GB), :], gsem[b])

    def wait_gather(b):
        pltpu.make_async_copy(
            t8_hbm.at[pl.ds(0, TOK), :], rows[b], gsem[b]).wait()

    def fire_store(s, b):
        pltpu.async_copy(
            rows[b], out_hbm.at[srow0 + s, :, pl.ds(0, D_MODEL)], ssem[b])

    def wait_store(b):
        pltpu.make_async_copy(
            rows[b], out_hbm.at[0, :, pl.ds(0, D_MODEL)], ssem[b]).wait()

    fire_gather(0, 0)
    fire_gather(1, 1)

    def body(i, carry):
        for db in range(NBUF):
            s = NBUF * i + db
            b = db
            bn = (db + 2) % NBUF
            if db == 0:
                @pl.when(i >= 1)
                def _():
                    wait_store(bn)
            else:
                wait_store(bn)
            fire_gather(s + 2, bn)
            wait_gather(b)
            fire_store(s, b)
        return carry

    lax.fori_loop(0, NMAIN // NBUF, body, 0)

    for (s, b) in ((NMAIN, 0), (NMAIN + 1, 1)):
        wait_gather(b)
        fire_store(s, b)
    for b in (2, 0, 1):
        wait_store(b)


_K2 = pl.kernel(
    _k2_body,
    mesh=plsc.VectorSubcoreMesh(core_axis_name="c", subcore_axis_name="s"),
    out_type=jax.ShapeDtypeStruct((SEQ, TOK, 2 * D_MODEL), jnp.float32),
    scratch_types=[
        pltpu.VMEM((ROWS_W, TOK), jnp.int32),
        pltpu.VMEM((TOK, D_MODEL), jnp.float32),
        pltpu.VMEM((TOK, D_MODEL), jnp.float32),
        pltpu.VMEM((TOK, D_MODEL), jnp.float32),
        pltpu.SemaphoreType.DMA,
        pltpu.SemaphoreType.DMA,
        pltpu.SemaphoreType.DMA,
        pltpu.SemaphoreType.DMA,
        pltpu.SemaphoreType.DMA,
        pltpu.SemaphoreType.DMA,
    ],
    compiler_params=pltpu.CompilerParams(use_tc_tiling_on_sc=False),
)


def kernel(x, table):
    t_t = table.T
    t8 = _K1(t_t, t_t)
    stg = t8.reshape(2 * P_ROWS, D_MODEL)
    v = x.astype(jnp.int32)
    idx = ((v >> 13) << 13) | ((v & 4095) << 1) | ((v >> 12) & 1)
    out_wide = _K2(idx, stg)
    return out_wide[:, :, :D_MODEL]


# speedup vs baseline: 2.4629x; 1.0571x over previous
import math

import jax
import jax.numpy as jnp
from jax import lax
from jax.experimental import pallas as pl
from jax.experimental.pallas import tpu as pltpu
from jax.experimental.pallas import tpu_sc as plsc

VOCAB_N = 1000000
D_MODEL = 64
SCALE = math.sqrt(D_MODEL)
NC, NS = 2, 16
NW = NC * NS
SEQ, TOK = 4096, 200
ROWS_W = SEQ // NW
NBUF = 3
NMAIN = (ROWS_W // NBUF) * NBUF
GA, GB = 128, TOK - 128
VB = 8192
LB = 13


K1_GRID = (VOCAB_N + 2 * VB - 1) // (2 * VB)
P_ROWS = K1_GRID * VB
N_INBLK = (VOCAB_N + VB - 1) // VB - 1


def _k1_body(tl_ref, th_ref, out_ref):
    lo = jnp.transpose(tl_ref[...], (1, 0)) * SCALE
    hi = jnp.transpose(th_ref[...], (1, 0)) * SCALE
    out_ref[...] = jnp.concatenate([lo, hi], axis=1)


_K1 = pl.pallas_call(
    _k1_body,
    grid=(K1_GRID,),
    in_specs=[
        pl.BlockSpec((D_MODEL, VB), lambda i: (0, jnp.minimum(2 * i, N_INBLK))),
        pl.BlockSpec((D_MODEL, VB),
                     lambda i: (0, jnp.minimum(2 * i + 1, N_INBLK))),
    ],
    out_specs=pl.BlockSpec((VB, 2 * D_MODEL), lambda i: (i, 0)),
    out_shape=jax.ShapeDtypeStruct((P_ROWS, 2 * D_MODEL), jnp.float32),
)


def _k2_body(x_hbm, t8_hbm, out_hbm, idx_v, r0, r1, r2, g0, g1, g2, s0, s1, s2):
    rows = (r0, r1, r2)
    gsem = (g0, g1, g2)
    ssem = (s0, s1, s2)
    wid = lax.axis_index("s") * NC + lax.axis_index("c")
    srow0 = wid * ROWS_W

    pltpu.sync_copy(x_hbm.at[pl.ds(srow0, ROWS_W), :], idx_v)

    def fire_gather(s, b):
        pltpu.async_copy(
            t8_hbm.at[idx_v.at[s, pl.ds(0, GA)]],
            rows[b].at[pl.ds(0, GA), :], gsem[b])
        pltpu.async_copy(
            t8_hbm.at[idx_v.at[s, pl.ds(GA, GB)]],
            rows[b].at[pl.ds(GA, GB), :], gsem[b])

    def wait_gather(b):
        pltpu.make_async_copy(
            t8_hbm.at[pl.ds(0, TOK), :], rows[b], gsem[b]).wait()

    def fire_store(s, b):
        pltpu.async_copy(
            rows[b], out_hbm.at[srow0 + s, :, pl.ds(0, D_MODEL)], ssem[b])

    def wait_store(b):
        pltpu.make_async_copy(
            rows[b], out_hbm.at[0, :, pl.ds(0, D_MODEL)], ssem[b]).wait()

    fire_gather(0, 0)
    fire_gather(1, 1)

    def body(i, carry):
        for db in range(NBUF):
            s = NBUF * i + db
            b = db
            bn = (db + 2) % NBUF
            if db == 0:
                @pl.when(i >= 1)
                def _():
                    wait_store(bn)
            else:
                wait_store(bn)
            fire_gather(s + 2, bn)
            wait_gather(b)
            fire_store(s, b)
        return carry

    lax.fori_loop(0, NMAIN // NBUF, body, 0)

    for (s, b) in ((NMAIN, 0), (NMAIN + 1, 1)):
        wait_gather(b)
        fire_store(s, b)
    for b in (2, 0, 1):
        wait_store(b)


_K2 = pl.kernel(
    _k2_body,
    mesh=plsc.VectorSubcoreMesh(core_axis_name="c", subcore_axis_name="s"),
    out_type=jax.ShapeDtypeStruct((SEQ, TOK, 2 * D_MODEL), jnp.float32),
    scratch_types=[
        pltpu.VMEM((ROWS_W, TOK), jnp.int32),
        pltpu.VMEM((TOK, D_MODEL), jnp.float32),
        pltpu.VMEM((TOK, D_MODEL), jnp.float32),
        pltpu.VMEM((TOK, D_MODEL), jnp.float32),
        pltpu.SemaphoreType.DMA,
        pltpu.SemaphoreType.DMA,
        pltpu.SemaphoreType.DMA,
        pltpu.SemaphoreType.DMA,
        pltpu.SemaphoreType.DMA,
        pltpu.SemaphoreType.DMA,
    ],
    compiler_params=pltpu.CompilerParams(use_tc_tiling_on_sc=False),
)


def kernel(x, table):
    t_t = table.T
    t8 = _K1(t_t, t_t)
    stg = t8.reshape(2 * P_ROWS, D_MODEL)
    v = x.astype(jnp.int32)
    idx = ((v >> (LB + 1)) << (LB + 1)) | ((v & (VB - 1)) << 1) | ((v >> LB) & 1)
    out_wide = _K2(idx, stg)
    return out_wide[:, :, :D_MODEL]


# speedup vs baseline: 2.5121x; 1.0200x over previous
import math

import jax
import jax.numpy as jnp
from jax import lax
from jax.experimental import pallas as pl
from jax.experimental.pallas import tpu as pltpu
from jax.experimental.pallas import tpu_sc as plsc

VOCAB_N = 1000000
D_MODEL = 64
SCALE = math.sqrt(D_MODEL)
NC, NS = 2, 16
NW = NC * NS
SEQ, TOK = 4096, 200
ROWS_W = SEQ // NW
NBUF = 3
NMAIN = (ROWS_W // NBUF) * NBUF
GA, GB = 128, TOK - 128
VB = 16384
LB = 14


K1_GRID = (VOCAB_N + 2 * VB - 1) // (2 * VB)
P_ROWS = K1_GRID * VB
N_INBLK = (VOCAB_N + VB - 1) // VB - 1


def _k1_body(tl_ref, th_ref, out_ref):
    lo = jnp.transpose(tl_ref[...], (1, 0)) * SCALE
    hi = jnp.transpose(th_ref[...], (1, 0)) * SCALE
    out_ref[...] = jnp.concatenate([lo, hi], axis=1)


_K1 = pl.pallas_call(
    _k1_body,
    grid=(K1_GRID,),
    in_specs=[
        pl.BlockSpec((D_MODEL, VB), lambda i: (0, jnp.minimum(2 * i, N_INBLK))),
        pl.BlockSpec((D_MODEL, VB),
                     lambda i: (0, jnp.minimum(2 * i + 1, N_INBLK))),
    ],
    out_specs=pl.BlockSpec((VB, 2 * D_MODEL), lambda i: (i, 0)),
    out_shape=jax.ShapeDtypeStruct((P_ROWS, 2 * D_MODEL), jnp.float32),
)


def _k2_body(x_hbm, t8_hbm, out_hbm, idx_v, r0, r1, r2, g0, g1, g2, s0, s1, s2):
    rows = (r0, r1, r2)
    gsem = (g0, g1, g2)
    ssem = (s0, s1, s2)
    wid = lax.axis_index("s") * NC + lax.axis_index("c")
    srow0 = wid * ROWS_W

    pltpu.sync_copy(x_hbm.at[pl.ds(srow0, ROWS_W), :], idx_v)

    def fire_gather(s, b):
        pltpu.async_copy(
            t8_hbm.at[idx_v.at[s, pl.ds(0, GA)]],
            rows[b].at[pl.ds(0, GA), :], gsem[b])
        pltpu.async_copy(
            t8_hbm.at[idx_v.at[s, pl.ds(GA, GB)]],
            rows[b].at[pl.ds(GA, GB), :], gsem[b])

    def wait_gather(b):
        pltpu.make_async_copy(
            t8_hbm.at[pl.ds(0, TOK), :], rows[b], gsem[b]).wait()

    def fire_store(s, b):
        pltpu.async_copy(
            rows[b], out_hbm.at[srow0 + s, :, pl.ds(0, D_MODEL)], ssem[b])

    def wait_store(b):
        pltpu.make_async_copy(
            rows[b], out_hbm.at[0, :, pl.ds(0, D_MODEL)], ssem[b]).wait()

    fire_gather(0, 0)
    fire_gather(1, 1)

    def body(i, carry):
        for db in range(NBUF):
            s = NBUF * i + db
            b = db
            bn = (db + 2) % NBUF
            if db == 0:
                @pl.when(i >= 1)
                def _():
                    wait_store(bn)
            else:
                wait_store(bn)
            fire_gather(s + 2, bn)
            wait_gather(b)
            fire_store(s, b)
        return carry

    lax.fori_loop(0, NMAIN // NBUF, body, 0)

    for (s, b) in ((NMAIN, 0), (NMAIN + 1, 1)):
        wait_gather(b)
        fire_store(s, b)
    for b in (2, 0, 1):
        wait_store(b)


_K2 = pl.kernel(
    _k2_body,
    mesh=plsc.VectorSubcoreMesh(core_axis_name="c", subcore_axis_name="s"),
    out_type=jax.ShapeDtypeStruct((SEQ, TOK, 2 * D_MODEL), jnp.float32),
    scratch_types=[
        pltpu.VMEM((ROWS_W, TOK), jnp.int32),
        pltpu.VMEM((TOK, D_MODEL), jnp.float32),
        pltpu.VMEM((TOK, D_MODEL), jnp.float32),
        pltpu.VMEM((TOK, D_MODEL), jnp.float32),
        pltpu.SemaphoreType.DMA,
        pltpu.SemaphoreType.DMA,
        pltpu.SemaphoreType.DMA,
        pltpu.SemaphoreType.DMA,
        pltpu.SemaphoreType.DMA,
        pltpu.SemaphoreType.DMA,
    ],
    compiler_params=pltpu.CompilerParams(use_tc_tiling_on_sc=False),
)


def kernel(x, table):
    t_t = table.T
    t8 = _K1(t_t, t_t)
    stg = t8.reshape(2 * P_ROWS, D_MODEL)
    v = x.astype(jnp.int32)
    idx = ((v >> (LB + 1)) << (LB + 1)) | ((v & (VB - 1)) << 1) | ((v >> LB) & 1)
    out_wide = _K2(idx, stg)
    return out_wide[:, :, :D_MODEL]
